# baseline (speedup 1.0000x reference)
import functools
import math

import jax
import jax.numpy as jnp
from jax import lax
from jax.experimental import pallas as pl
from jax.experimental.pallas import tpu as pltpu

NB = 32
C_IN = 64
C_X = 32
C_ALL = C_IN + C_X
PT = 128
RT = PT * NB
TWO_PI = 2.0 * math.pi


def _bf(v):
    return v.astype(jnp.bfloat16)


def _k1_body(gf_ref, gx_ref, gx0_ref, b8_ref, b6_ref, k4_ref,
             mlpwt_ref, mlpb_ref, acc_ref):
    gx = gx_ref[...]
    gx0 = gx0_ref[...]
    xrel = gx - gx0
    xdis = jnp.sqrt(jnp.sum(xrel * xrel, axis=-1, keepdims=True))
    v8 = jnp.concatenate([TWO_PI * gx0, TWO_PI * xrel], axis=-1)
    z = jnp.dot(_bf(v8), b8_ref[...], preferred_element_type=jnp.float32)
    z = z + _bf(TWO_PI * xdis).astype(jnp.float32) * b6_ref[...]
    scf = jnp.concatenate([jnp.sin(z), jnp.cos(z)], axis=-1)
    xf = jnp.dot(_bf(scf), mlpwt_ref[...],
                 preferred_element_type=jnp.float32) + mlpb_ref[...]
    rhs = jnp.concatenate([gf_ref[...], _bf(xf)], axis=-1)

    pr = jnp.dot(_bf(xrel), k4_ref[...], preferred_element_type=jnp.float32)
    ri = lax.broadcasted_iota(jnp.int32, (RT, NB), 0)
    ci = lax.broadcasted_iota(jnp.int32, (RT, NB), 1)
    onepad = jnp.where((ri % NB == 0) & (ci == 0), 1.0, 0.0)
    pr = jnp.maximum(pr + onepad, 0.0)
    pr3 = pr.reshape(PT, NB, NB)
    s = jnp.sum(pr3, axis=1, keepdims=True)
    perm = _bf((pr3 / (s + 1e-6)).reshape(RT, NB))

    m_r = lax.broadcasted_iota(jnp.int32, (256, 256), 0)
    m_c = lax.broadcasted_iota(jnp.int32, (256, 256), 1)
    mask = jnp.where((m_r // NB) == (m_c // NB), 1.0, 0.0).astype(jnp.bfloat16)
    for ch in range(RT // 256):
        pc = perm[ch * 256:(ch + 1) * 256, :]
        bd = jnp.concatenate([pc] * 8, axis=1) * mask
        rc = rhs[ch * 256:(ch + 1) * 256, :]
        gtc = lax.dot_general(bd, rc, (((0,), (0,)), ((), ())),
                              preferred_element_type=jnp.float32)
        acc_ref[ch * 256:(ch + 1) * 256, :] = _bf(gtc)


def _k1b_body(gflat_ref, w2_ref, convb_ref, outt_ref, psum_ref, psumsq_ref):
    outt = lax.dot_general(w2_ref[...], gflat_ref[...], (((0,), (1,)), ((), ())),
                           preferred_element_type=jnp.float32)
    outt = outt + convb_ref[...]
    outt_ref[...] = outt
    psum_ref[0] = jnp.sum(outt, axis=1, keepdims=True)
    psumsq_ref[0] = jnp.sum(outt * outt, axis=1, keepdims=True)


def _k2_body(pre_ref, psum_ref, psumsq_ref, gamma_ref, beta_ref, out_ref, *, total):
    s = jnp.sum(psum_ref[...], axis=0)
    ss = jnp.sum(psumsq_ref[...], axis=0)
    mean = s / total
    var = ss / total - mean * mean
    scale = gamma_ref[...] * lax.rsqrt(var + 1e-5)
    shift = beta_ref[...] - mean * scale
    out_ref[0] = pre_ref[...] * scale + shift


def kernel(x, feature, neigh_indexs, B, kernels, mlp_w, mlp_b, conv_w, conv_b, bn_gamma, bn_beta):
    BS, C, N = feature.shape
    P = BS * N
    R = P * NB
    T = P // PT

    ftab = jnp.transpose(feature, (0, 2, 1)).reshape(P, C).astype(jnp.bfloat16)
    ctab = jnp.concatenate(
        [jnp.transpose(x, (0, 2, 1)).reshape(P, 3), jnp.zeros((P, 1), jnp.float32)], axis=1)
    base = (jnp.arange(BS, dtype=jnp.int32) * N).reshape(BS, 1, 1)
    ni32 = neigh_indexs.astype(jnp.int32)
    flat_ni = (ni32 + base).reshape(-1)
    flat_ni0 = jnp.broadcast_to(ni32[:, :, 0:1] + base, (BS, N, NB)).reshape(-1)

    zero_row = jnp.zeros((1, 32), jnp.float32)
    b8 = jnp.concatenate([B[0:3], zero_row, B[3:6], zero_row], axis=0).astype(jnp.bfloat16)
    b6 = B[6:7].astype(jnp.bfloat16).astype(jnp.float32)
    k4 = jnp.concatenate([kernels, zero_row], axis=0).astype(jnp.bfloat16)
    mlpwt = mlp_w.T.astype(jnp.bfloat16)
    mlpb = mlp_b.reshape(1, C_X)
    w2 = conv_w.reshape(64, C_ALL, NB).transpose(2, 1, 0).reshape(NB * C_ALL, 64).astype(jnp.bfloat16)
    convb = conv_b.reshape(64, 1)
    gamma = bn_gamma.reshape(64, 1)
    beta = bn_beta.reshape(64, 1)

    gf = jnp.take(ftab, flat_ni, axis=0)
    gx = jnp.take(ctab, flat_ni, axis=0)
    gx0 = jnp.take(ctab, flat_ni0, axis=0)

    full = lambda shape: pl.BlockSpec(shape, lambda t: (0,) * len(shape))
    gt = pl.pallas_call(
        _k1_body,
        grid=(T,),
        in_specs=[
            pl.BlockSpec((RT, C_IN), lambda t: (t, 0)),
            pl.BlockSpec((RT, 4), lambda t: (t, 0)),
            pl.BlockSpec((RT, 4), lambda t: (t, 0)),
            full((8, 32)), full((1, 32)), full((4, 32)),
            full((64, C_X)), full((1, C_X)),
        ],
        out_specs=pl.BlockSpec((RT, C_ALL), lambda t: (t, 0)),
        out_shape=jax.ShapeDtypeStruct((R, C_ALL), jnp.bfloat16),
    )(gf, gx, gx0, b8, b6, k4, mlpwt, mlpb)

    gflat = gt.reshape(P, NB * C_ALL)

    outt, psum, psumsq = pl.pallas_call(
        _k1b_body,
        grid=(T,),
        in_specs=[
            pl.BlockSpec((PT, NB * C_ALL), lambda t: (t, 0)),
            full((NB * C_ALL, 64)), full((64, 1)),
        ],
        out_specs=[
            pl.BlockSpec((64, PT), lambda t: (0, t)),
            pl.BlockSpec((1, 64, 1), lambda t: (t, 0, 0)),
            pl.BlockSpec((1, 64, 1), lambda t: (t, 0, 0)),
        ],
        out_shape=[
            jax.ShapeDtypeStruct((64, P), jnp.float32),
            jax.ShapeDtypeStruct((T, 64, 1), jnp.float32),
            jax.ShapeDtypeStruct((T, 64, 1), jnp.float32),
        ],
    )(gflat, w2, convb)

    NT2 = 1024
    T2 = P // NT2
    out = pl.pallas_call(
        functools.partial(_k2_body, total=float(P)),
        grid=(T2,),
        in_specs=[
            pl.BlockSpec((64, NT2), lambda t: (0, t)),
            pl.BlockSpec((T, 64, 1), lambda t: (0, 0, 0)),
            pl.BlockSpec((T, 64, 1), lambda t: (0, 0, 0)),
            pl.BlockSpec((64, 1), lambda t: (0, 0)),
            pl.BlockSpec((64, 1), lambda t: (0, 0)),
        ],
        out_specs=pl.BlockSpec((1, 64, NT2), lambda t: (t * NT2 // N, 0, (t * NT2 % N) // NT2)),
        out_shape=jax.ShapeDtypeStruct((BS, 64, N), jnp.float32),
    )(outt, psum, psumsq, gamma, beta)
    return out


# speedup vs baseline: 2.3338x; 2.3338x over previous
import functools
import math

import jax
import jax.numpy as jnp
from jax import lax
from jax.experimental import pallas as pl
from jax.experimental.pallas import tpu as pltpu
from jax.experimental.pallas import tpu_sc as plsc

NB = 32
C_IN = 64
C_X = 32
C_ALL = C_IN + C_X
PT = 128
RT = PT * NB
TWO_PI = 2.0 * math.pi


def _bf(v):
    return v.astype(jnp.bfloat16)


_SC_CH = 128
_SC_NCH = 4


def _make_sc_gather(R, P, fw):
    NW = 32
    bpw = R // NW
    step = _SC_CH * _SC_NCH
    nsteps = bpw // step
    mesh = plsc.VectorSubcoreMesh(core_axis_name="c", subcore_axis_name="s")

    @functools.partial(
        pl.kernel, mesh=mesh,
        compiler_params=pltpu.CompilerParams(use_tc_tiling_on_sc=False),
        out_type=[
            jax.ShapeDtypeStruct((R, fw), jnp.int32),
            jax.ShapeDtypeStruct((R, 16), jnp.float32),
            jax.ShapeDtypeStruct((R, 16), jnp.float32),
        ],
        scratch_types=[
            pltpu.VMEM((step,), jnp.int32),
            pltpu.VMEM((step,), jnp.int32),
            pltpu.VMEM((step, fw), jnp.int32),
            pltpu.VMEM((step, 16), jnp.float32),
            pltpu.VMEM((step, 16), jnp.float32),
            pltpu.SemaphoreType.DMA,
        ],
    )
    def sc_gather(ni_hbm, ni0_hbm, ftab_hbm, ctab_hbm,
                  gf_hbm, gx_hbm, gx0_hbm,
                  idx_v, idx0_v, frows_v, crows_v, c0rows_v, sem):
        wid = lax.axis_index("s") * 2 + lax.axis_index("c")
        wbase = wid * bpw

        def body(g, carry):
            base = wbase + g * step
            pltpu.sync_copy(ni_hbm.at[pl.ds(base, step)], idx_v)
            pltpu.sync_copy(ni0_hbm.at[pl.ds(base, step)], idx0_v)
            waits = []
            for k in range(_SC_NCH):
                sl = pl.ds(k * _SC_CH, _SC_CH)
                waits.append(pltpu.async_copy(
                    ftab_hbm.at[idx_v.at[sl]], frows_v.at[sl], sem))
                waits.append(pltpu.async_copy(
                    ctab_hbm.at[idx_v.at[sl]], crows_v.at[sl], sem))
                waits.append(pltpu.async_copy(
                    ctab_hbm.at[idx0_v.at[sl]], c0rows_v.at[sl], sem))
            for h in waits:
                h.wait()
            pltpu.sync_copy(frows_v, gf_hbm.at[pl.ds(base, step)])
            pltpu.sync_copy(crows_v, gx_hbm.at[pl.ds(base, step)])
            pltpu.sync_copy(c0rows_v, gx0_hbm.at[pl.ds(base, step)])
            return carry

        lax.fori_loop(0, nsteps, body, 0)

    return sc_gather


def _k1_body(gf_ref, gx_ref, gx0_ref, b8_ref, b6_ref, k4_ref,
             mlpwt_ref, mlpb_ref, acc_ref):
    gx = gx_ref[:, 0:4]
    gx0 = gx0_ref[:, 0:4]
    xrel = gx - gx0
    xdis = jnp.sqrt(jnp.sum(xrel * xrel, axis=-1, keepdims=True))
    v8 = jnp.concatenate([TWO_PI * gx0, TWO_PI * xrel], axis=-1)
    z = jnp.dot(_bf(v8), b8_ref[...], preferred_element_type=jnp.float32)
    z = z + _bf(TWO_PI * xdis).astype(jnp.float32) * b6_ref[...]
    scf = jnp.concatenate([jnp.sin(z), jnp.cos(z)], axis=-1)
    xf = jnp.dot(_bf(scf), mlpwt_ref[...],
                 preferred_element_type=jnp.float32) + mlpb_ref[...]
    rhs = jnp.concatenate([gf_ref[...], _bf(xf)], axis=-1)

    pr = jnp.dot(_bf(xrel), k4_ref[...], preferred_element_type=jnp.float32)
    ri = lax.broadcasted_iota(jnp.int32, (RT, NB), 0)
    ci = lax.broadcasted_iota(jnp.int32, (RT, NB), 1)
    onepad = jnp.where((ri % NB == 0) & (ci == 0), 1.0, 0.0)
    pr = jnp.maximum(pr + onepad, 0.0)
    pr3 = pr.reshape(PT, NB, NB)
    s = jnp.sum(pr3, axis=1, keepdims=True)
    perm = _bf((pr3 / (s + 1e-6)).reshape(RT, NB))

    m_r = lax.broadcasted_iota(jnp.int32, (256, 256), 0)
    m_c = lax.broadcasted_iota(jnp.int32, (256, 256), 1)
    mask = jnp.where((m_r // NB) == (m_c // NB), 1.0, 0.0).astype(jnp.bfloat16)
    for ch in range(RT // 256):
        pc = perm[ch * 256:(ch + 1) * 256, :]
        bd = jnp.concatenate([pc] * 8, axis=1) * mask
        rc = rhs[ch * 256:(ch + 1) * 256, :]
        gtc = lax.dot_general(bd, rc, (((0,), (0,)), ((), ())),
                              preferred_element_type=jnp.float32)
        acc_ref[ch * 256:(ch + 1) * 256, :] = _bf(gtc)


def _k1b_body(gflat_ref, w2_ref, convb_ref, outt_ref, psum_ref, psumsq_ref):
    outt = lax.dot_general(w2_ref[...], gflat_ref[...], (((0,), (1,)), ((), ())),
                           preferred_element_type=jnp.float32)
    outt = outt + convb_ref[...]
    outt_ref[...] = outt
    psum_ref[0] = jnp.sum(outt, axis=1, keepdims=True)
    psumsq_ref[0] = jnp.sum(outt * outt, axis=1, keepdims=True)


def _k2_body(pre_ref, psum_ref, psumsq_ref, gamma_ref, beta_ref, out_ref, *, total):
    s = jnp.sum(psum_ref[...], axis=0)
    ss = jnp.sum(psumsq_ref[...], axis=0)
    mean = s / total
    var = ss / total - mean * mean
    scale = gamma_ref[...] * lax.rsqrt(var + 1e-5)
    shift = beta_ref[...] - mean * scale
    out_ref[0] = pre_ref[...] * scale + shift


def kernel(x, feature, neigh_indexs, B, kernels, mlp_w, mlp_b, conv_w, conv_b, bn_gamma, bn_beta):
    BS, C, N = feature.shape
    P = BS * N
    R = P * NB
    T = P // PT

    ftab = jnp.transpose(feature, (0, 2, 1)).reshape(P, C).astype(jnp.bfloat16)
    ctab = jnp.concatenate(
        [jnp.transpose(x, (0, 2, 1)).reshape(P, 3), jnp.zeros((P, 13), jnp.float32)], axis=1)
    base = (jnp.arange(BS, dtype=jnp.int32) * N).reshape(BS, 1, 1)
    ni32 = neigh_indexs.astype(jnp.int32)
    flat_ni = (ni32 + base).reshape(-1)
    flat_ni0 = jnp.broadcast_to(ni32[:, :, 0:1] + base, (BS, N, NB)).reshape(-1)

    zero_row = jnp.zeros((1, 32), jnp.float32)
    b8 = jnp.concatenate([B[0:3], zero_row, B[3:6], zero_row], axis=0).astype(jnp.bfloat16)
    b6 = B[6:7].astype(jnp.bfloat16).astype(jnp.float32)
    k4 = jnp.concatenate([kernels, zero_row], axis=0).astype(jnp.bfloat16)
    mlpwt = mlp_w.T.astype(jnp.bfloat16)
    mlpb = mlp_b.reshape(1, C_X)
    w2 = conv_w.reshape(64, C_ALL, NB).transpose(2, 1, 0).reshape(NB * C_ALL, 64).astype(jnp.bfloat16)
    convb = conv_b.reshape(64, 1)
    gamma = bn_gamma.reshape(64, 1)
    beta = bn_beta.reshape(64, 1)

    ftab_p = lax.bitcast_convert_type(ftab.reshape(P, C // 2, 2), jnp.int32)
    gfp, gx, gx0 = _make_sc_gather(R, P, C // 2)(flat_ni, flat_ni0, ftab_p, ctab)
    gf = lax.bitcast_convert_type(gfp, jnp.bfloat16).reshape(R, C)

    full = lambda shape: pl.BlockSpec(shape, lambda t: (0,) * len(shape))
    gt = pl.pallas_call(
        _k1_body,
        grid=(T,),
        in_specs=[
            pl.BlockSpec((RT, C_IN), lambda t: (t, 0)),
            pl.BlockSpec((RT, 16), lambda t: (t, 0)),
            pl.BlockSpec((RT, 16), lambda t: (t, 0)),
            full((8, 32)), full((1, 32)), full((4, 32)),
            full((64, C_X)), full((1, C_X)),
        ],
        out_specs=pl.BlockSpec((RT, C_ALL), lambda t: (t, 0)),
        out_shape=jax.ShapeDtypeStruct((R, C_ALL), jnp.bfloat16),
    )(gf, gx, gx0, b8, b6, k4, mlpwt, mlpb)

    gflat = gt.reshape(P, NB * C_ALL)

    outt, psum, psumsq = pl.pallas_call(
        _k1b_body,
        grid=(T,),
        in_specs=[
            pl.BlockSpec((PT, NB * C_ALL), lambda t: (t, 0)),
            full((NB * C_ALL, 64)), full((64, 1)),
        ],
        out_specs=[
            pl.BlockSpec((64, PT), lambda t: (0, t)),
            pl.BlockSpec((1, 64, 1), lambda t: (t, 0, 0)),
            pl.BlockSpec((1, 64, 1), lambda t: (t, 0, 0)),
        ],
        out_shape=[
            jax.ShapeDtypeStruct((64, P), jnp.float32),
            jax.ShapeDtypeStruct((T, 64, 1), jnp.float32),
            jax.ShapeDtypeStruct((T, 64, 1), jnp.float32),
        ],
    )(gflat, w2, convb)

    NT2 = 1024
    T2 = P // NT2
    out = pl.pallas_call(
        functools.partial(_k2_body, total=float(P)),
        grid=(T2,),
        in_specs=[
            pl.BlockSpec((64, NT2), lambda t: (0, t)),
            pl.BlockSpec((T, 64, 1), lambda t: (0, 0, 0)),
            pl.BlockSpec((T, 64, 1), lambda t: (0, 0, 0)),
            pl.BlockSpec((64, 1), lambda t: (0, 0)),
            pl.BlockSpec((64, 1), lambda t: (0, 0)),
        ],
        out_specs=pl.BlockSpec((1, 64, NT2), lambda t: (t * NT2 // N, 0, (t * NT2 % N) // NT2)),
        out_shape=jax.ShapeDtypeStruct((BS, 64, N), jnp.float32),
    )(outt, psum, psumsq, gamma, beta)
    return out


# speedup vs baseline: 2.3629x; 1.0125x over previous
import functools
import math

import jax
import jax.numpy as jnp
from jax import lax
from jax.experimental import pallas as pl
from jax.experimental.pallas import tpu as pltpu
from jax.experimental.pallas import tpu_sc as plsc

NB = 32
C_IN = 64
C_X = 32
C_ALL = C_IN + C_X
PT = 128
RT = PT * NB
TWO_PI = 2.0 * math.pi


def _bf(v):
    return v.astype(jnp.bfloat16)


_SC_CH = 128
_SC_NCH = 4


def _make_sc_gather(R, P, fw):
    NW = 32
    bpw = R // NW
    step = _SC_CH * _SC_NCH
    nsteps = bpw // step
    mesh = plsc.VectorSubcoreMesh(core_axis_name="c", subcore_axis_name="s")

    @functools.partial(
        pl.kernel, mesh=mesh,
        compiler_params=pltpu.CompilerParams(use_tc_tiling_on_sc=False),
        out_type=[
            jax.ShapeDtypeStruct((R, fw), jnp.int32),
            jax.ShapeDtypeStruct((R, 16), jnp.float32),
            jax.ShapeDtypeStruct((R, 16), jnp.float32),
        ],
        scratch_types=[
            pltpu.VMEM((2, step), jnp.int32),
            pltpu.VMEM((2, step), jnp.int32),
            pltpu.VMEM((2, step, fw), jnp.int32),
            pltpu.VMEM((2, step, 16), jnp.float32),
            pltpu.VMEM((2, step, 16), jnp.float32),
            pltpu.SemaphoreType.DMA,
            pltpu.SemaphoreType.DMA,
            pltpu.SemaphoreType.DMA,
            pltpu.SemaphoreType.DMA,
        ],
    )
    def sc_gather(ni_hbm, ni0_hbm, ftab_hbm, ctab_hbm,
                  gf_hbm, gx_hbm, gx0_hbm,
                  idx_v, idx0_v, frows_v, crows_v, c0rows_v,
                  sg0, sg1, sw0, sw1):
        wid = lax.axis_index("s") * 2 + lax.axis_index("c")
        wbase = wid * bpw
        sgs = (sg0, sg1)
        sws = (sw0, sw1)

        def fire_gathers(g):
            b = g % 2
            base = wbase + g * step
            pltpu.sync_copy(ni_hbm.at[pl.ds(base, step)], idx_v.at[b])
            pltpu.sync_copy(ni0_hbm.at[pl.ds(base, step)], idx0_v.at[b])
            hs = []
            for k in range(_SC_NCH):
                sl = pl.ds(k * _SC_CH, _SC_CH)
                hs.append(pltpu.async_copy(
                    ftab_hbm.at[idx_v.at[b, sl]], frows_v.at[b, sl], sgs[b]))
                hs.append(pltpu.async_copy(
                    ctab_hbm.at[idx_v.at[b, sl]], crows_v.at[b, sl], sgs[b]))
                hs.append(pltpu.async_copy(
                    ctab_hbm.at[idx0_v.at[b, sl]], c0rows_v.at[b, sl], sgs[b]))
            return hs

        def fire_writes(g):
            b = g % 2
            base = wbase + g * step
            return [
                pltpu.async_copy(frows_v.at[b], gf_hbm.at[pl.ds(base, step)], sws[b]),
                pltpu.async_copy(crows_v.at[b], gx_hbm.at[pl.ds(base, step)], sws[b]),
                pltpu.async_copy(c0rows_v.at[b], gx0_hbm.at[pl.ds(base, step)], sws[b]),
            ]

        gh = {0: fire_gathers(0)}
        wh = {}
        for g in range(nsteps):
            if g + 1 < nsteps:
                if g - 1 in wh:
                    for h in wh.pop(g - 1):
                        h.wait()
                gh[g + 1] = fire_gathers(g + 1)
            for h in gh.pop(g):
                h.wait()
            wh[g] = fire_writes(g)
        for hs in wh.values():
            for h in hs:
                h.wait()

    return sc_gather


def _k1_body(gf_ref, gx_ref, gx0_ref, b8_ref, b6_ref, k4_ref,
             mlpwt_ref, mlpb_ref, acc_ref):
    gx = gx_ref[:, 0:4]
    gx0 = gx0_ref[:, 0:4]
    xrel = gx - gx0
    xdis = jnp.sqrt(jnp.sum(xrel * xrel, axis=-1, keepdims=True))
    v8 = jnp.concatenate([TWO_PI * gx0, TWO_PI * xrel], axis=-1)
    z = jnp.dot(_bf(v8), b8_ref[...], preferred_element_type=jnp.float32)
    z = z + _bf(TWO_PI * xdis).astype(jnp.float32) * b6_ref[...]
    scf = jnp.concatenate([jnp.sin(z), jnp.cos(z)], axis=-1)
    xf = jnp.dot(_bf(scf), mlpwt_ref[...],
                 preferred_element_type=jnp.float32) + mlpb_ref[...]
    rhs = jnp.concatenate([gf_ref[...], _bf(xf)], axis=-1)

    pr = jnp.dot(_bf(xrel), k4_ref[...], preferred_element_type=jnp.float32)
    ri = lax.broadcasted_iota(jnp.int32, (RT, NB), 0)
    ci = lax.broadcasted_iota(jnp.int32, (RT, NB), 1)
    onepad = jnp.where((ri % NB == 0) & (ci == 0), 1.0, 0.0)
    pr = jnp.maximum(pr + onepad, 0.0)
    pr3 = pr.reshape(PT, NB, NB)
    s = jnp.sum(pr3, axis=1, keepdims=True)
    perm = _bf((pr3 / (s + 1e-6)).reshape(RT, NB))

    m_r = lax.broadcasted_iota(jnp.int32, (256, 256), 0)
    m_c = lax.broadcasted_iota(jnp.int32, (256, 256), 1)
    mask = jnp.where((m_r // NB) == (m_c // NB), 1.0, 0.0).astype(jnp.bfloat16)
    for ch in range(RT // 256):
        pc = perm[ch * 256:(ch + 1) * 256, :]
        bd = jnp.concatenate([pc] * 8, axis=1) * mask
        rc = rhs[ch * 256:(ch + 1) * 256, :]
        gtc = lax.dot_general(bd, rc, (((0,), (0,)), ((), ())),
                              preferred_element_type=jnp.float32)
        acc_ref[ch * 256:(ch + 1) * 256, :] = _bf(gtc)


def _k1b_body(gflat_ref, w2_ref, convb_ref, outt_ref, psum_ref, psumsq_ref):
    outt = lax.dot_general(w2_ref[...], gflat_ref[...], (((0,), (1,)), ((), ())),
                           preferred_element_type=jnp.float32)
    outt = outt + convb_ref[...]
    outt_ref[...] = outt
    psum_ref[0] = jnp.sum(outt, axis=1, keepdims=True)
    psumsq_ref[0] = jnp.sum(outt * outt, axis=1, keepdims=True)


def _k2_body(pre_ref, psum_ref, psumsq_ref, gamma_ref, beta_ref, out_ref, *, total):
    s = jnp.sum(psum_ref[...], axis=0)
    ss = jnp.sum(psumsq_ref[...], axis=0)
    mean = s / total
    var = ss / total - mean * mean
    scale = gamma_ref[...] * lax.rsqrt(var + 1e-5)
    shift = beta_ref[...] - mean * scale
    out_ref[0] = pre_ref[...] * scale + shift


def kernel(x, feature, neigh_indexs, B, kernels, mlp_w, mlp_b, conv_w, conv_b, bn_gamma, bn_beta):
    BS, C, N = feature.shape
    P = BS * N
    R = P * NB
    T = P // PT

    ftab = jnp.transpose(feature, (0, 2, 1)).reshape(P, C).astype(jnp.bfloat16)
    ctab = jnp.concatenate(
        [jnp.transpose(x, (0, 2, 1)).reshape(P, 3), jnp.zeros((P, 13), jnp.float32)], axis=1)
    base = (jnp.arange(BS, dtype=jnp.int32) * N).reshape(BS, 1, 1)
    ni32 = neigh_indexs.astype(jnp.int32)
    flat_ni = (ni32 + base).reshape(-1)
    flat_ni0 = jnp.broadcast_to(ni32[:, :, 0:1] + base, (BS, N, NB)).reshape(-1)

    zero_row = jnp.zeros((1, 32), jnp.float32)
    b8 = jnp.concatenate([B[0:3], zero_row, B[3:6], zero_row], axis=0).astype(jnp.bfloat16)
    b6 = B[6:7].astype(jnp.bfloat16).astype(jnp.float32)
    k4 = jnp.concatenate([kernels, zero_row], axis=0).astype(jnp.bfloat16)
    mlpwt = mlp_w.T.astype(jnp.bfloat16)
    mlpb = mlp_b.reshape(1, C_X)
    w2 = conv_w.reshape(64, C_ALL, NB).transpose(2, 1, 0).reshape(NB * C_ALL, 64).astype(jnp.bfloat16)
    convb = conv_b.reshape(64, 1)
    gamma = bn_gamma.reshape(64, 1)
    beta = bn_beta.reshape(64, 1)

    ftab_p = lax.bitcast_convert_type(ftab.reshape(P, C // 2, 2), jnp.int32)
    gfp, gx, gx0 = _make_sc_gather(R, P, C // 2)(flat_ni, flat_ni0, ftab_p, ctab)
    gf = lax.bitcast_convert_type(gfp, jnp.bfloat16).reshape(R, C)

    full = lambda shape: pl.BlockSpec(shape, lambda t: (0,) * len(shape))
    gt = pl.pallas_call(
        _k1_body,
        grid=(T,),
        in_specs=[
            pl.BlockSpec((RT, C_IN), lambda t: (t, 0)),
            pl.BlockSpec((RT, 16), lambda t: (t, 0)),
            pl.BlockSpec((RT, 16), lambda t: (t, 0)),
            full((8, 32)), full((1, 32)), full((4, 32)),
            full((64, C_X)), full((1, C_X)),
        ],
        out_specs=pl.BlockSpec((RT, C_ALL), lambda t: (t, 0)),
        out_shape=jax.ShapeDtypeStruct((R, C_ALL), jnp.bfloat16),
    )(gf, gx, gx0, b8, b6, k4, mlpwt, mlpb)

    gflat = gt.reshape(P, NB * C_ALL)

    outt, psum, psumsq = pl.pallas_call(
        _k1b_body,
        grid=(T,),
        in_specs=[
            pl.BlockSpec((PT, NB * C_ALL), lambda t: (t, 0)),
            full((NB * C_ALL, 64)), full((64, 1)),
        ],
        out_specs=[
            pl.BlockSpec((64, PT), lambda t: (0, t)),
            pl.BlockSpec((1, 64, 1), lambda t: (t, 0, 0)),
            pl.BlockSpec((1, 64, 1), lambda t: (t, 0, 0)),
        ],
        out_shape=[
            jax.ShapeDtypeStruct((64, P), jnp.float32),
            jax.ShapeDtypeStruct((T, 64, 1), jnp.float32),
            jax.ShapeDtypeStruct((T, 64, 1), jnp.float32),
        ],
    )(gflat, w2, convb)

    NT2 = 1024
    T2 = P // NT2
    out = pl.pallas_call(
        functools.partial(_k2_body, total=float(P)),
        grid=(T2,),
        in_specs=[
            pl.BlockSpec((64, NT2), lambda t: (0, t)),
            pl.BlockSpec((T, 64, 1), lambda t: (0, 0, 0)),
            pl.BlockSpec((T, 64, 1), lambda t: (0, 0, 0)),
            pl.BlockSpec((64, 1), lambda t: (0, 0)),
            pl.BlockSpec((64, 1), lambda t: (0, 0)),
        ],
        out_specs=pl.BlockSpec((1, 64, NT2), lambda t: (t * NT2 // N, 0, (t * NT2 % N) // NT2)),
        out_shape=jax.ShapeDtypeStruct((BS, 64, N), jnp.float32),
    )(outt, psum, psumsq, gamma, beta)
    return out


# speedup vs baseline: 2.9567x; 1.2513x over previous
import functools
import math

import jax
import jax.numpy as jnp
from jax import lax
from jax.experimental import pallas as pl
from jax.experimental.pallas import tpu as pltpu
from jax.experimental.pallas import tpu_sc as plsc

NB = 32
C_IN = 64
C_X = 32
C_ALL = C_IN + C_X
PT = 128
RT = PT * NB
TWO_PI = 2.0 * math.pi


def _bf(v):
    return v.astype(jnp.bfloat16)


_SC_CH = 128
_SC_NCH = 4


def _make_sc_gather(R, P, fw):
    NW = 32
    bpw = R // NW
    step = _SC_CH * _SC_NCH
    nsteps = bpw // step
    mesh = plsc.VectorSubcoreMesh(core_axis_name="c", subcore_axis_name="s")

    @functools.partial(
        pl.kernel, mesh=mesh,
        compiler_params=pltpu.CompilerParams(use_tc_tiling_on_sc=False),
        out_type=[
            jax.ShapeDtypeStruct((R, fw), jnp.bfloat16),
            jax.ShapeDtypeStruct((R, 16), jnp.float32),
            jax.ShapeDtypeStruct((R, 16), jnp.float32),
        ],
        scratch_types=[
            pltpu.VMEM((2, step), jnp.int32),
            pltpu.VMEM((2, step), jnp.int32),
            pltpu.VMEM((2, step, fw), jnp.bfloat16),
            pltpu.VMEM((2, step, 16), jnp.float32),
            pltpu.VMEM((2, step, 16), jnp.float32),
            pltpu.SemaphoreType.DMA,
            pltpu.SemaphoreType.DMA,
            pltpu.SemaphoreType.DMA,
            pltpu.SemaphoreType.DMA,
        ],
    )
    def sc_gather(ni_hbm, ni0_hbm, ftab_hbm, ctab_hbm,
                  gf_hbm, gx_hbm, gx0_hbm,
                  idx_v, idx0_v, frows_v, crows_v, c0rows_v,
                  sg0, sg1, sw0, sw1):
        wid = lax.axis_index("s") * 2 + lax.axis_index("c")
        wbase = wid * bpw
        sgs = (sg0, sg1)
        sws = (sw0, sw1)

        def fire_gathers(g):
            b = g % 2
            base = wbase + g * step
            pltpu.sync_copy(ni_hbm.at[pl.ds(base, step)], idx_v.at[b])
            pltpu.sync_copy(ni0_hbm.at[pl.ds(base, step)], idx0_v.at[b])
            hs = []
            for k in range(_SC_NCH):
                sl = pl.ds(k * _SC_CH, _SC_CH)
                hs.append(pltpu.async_copy(
                    ftab_hbm.at[idx_v.at[b, sl]], frows_v.at[b, sl], sgs[b]))
                hs.append(pltpu.async_copy(
                    ctab_hbm.at[idx_v.at[b, sl]], crows_v.at[b, sl], sgs[b]))
                hs.append(pltpu.async_copy(
                    ctab_hbm.at[idx0_v.at[b, sl]], c0rows_v.at[b, sl], sgs[b]))
            return hs

        def fire_writes(g):
            b = g % 2
            base = wbase + g * step
            return [
                pltpu.async_copy(frows_v.at[b], gf_hbm.at[pl.ds(base, step)], sws[b]),
                pltpu.async_copy(crows_v.at[b], gx_hbm.at[pl.ds(base, step)], sws[b]),
                pltpu.async_copy(c0rows_v.at[b], gx0_hbm.at[pl.ds(base, step)], sws[b]),
            ]

        gh = {0: fire_gathers(0)}
        wh = {}
        for g in range(nsteps):
            if g + 1 < nsteps:
                if g - 1 in wh:
                    for h in wh.pop(g - 1):
                        h.wait()
                gh[g + 1] = fire_gathers(g + 1)
            for h in gh.pop(g):
                h.wait()
            wh[g] = fire_writes(g)
        for hs in wh.values():
            for h in hs:
                h.wait()

    return sc_gather


def _k1_body(gf_ref, gx_ref, gx0_ref, b8_ref, b6_ref, k4_ref,
             mlpwt_ref, mlpb_ref, onep_ref, rep_ref, mask_ref, acc_ref):
    gx = gx_ref[:, 0:4]
    gx0 = gx0_ref[:, 0:4]
    xrel = gx - gx0
    xdis = jnp.sqrt(jnp.sum(xrel * xrel, axis=-1, keepdims=True))
    v8 = jnp.concatenate([TWO_PI * gx0, TWO_PI * xrel], axis=-1)
    z = jnp.dot(_bf(v8), b8_ref[...], preferred_element_type=jnp.float32)
    z = z + _bf(TWO_PI * xdis).astype(jnp.float32) * b6_ref[...]
    scf = jnp.concatenate([jnp.sin(z), jnp.cos(z)], axis=-1)
    xf = jnp.dot(_bf(scf), mlpwt_ref[...],
                 preferred_element_type=jnp.float32) + mlpb_ref[...]
    rhs = jnp.concatenate([gf_ref[...], _bf(xf)], axis=-1)

    pr = jnp.dot(_bf(xrel), k4_ref[...], preferred_element_type=jnp.float32)
    pr3 = pr.reshape(PT, NB, NB) + onep_ref[...][None, :, :]
    pr3 = jnp.maximum(pr3, 0.0)
    s = jnp.sum(pr3, axis=1, keepdims=True)
    perm = _bf((pr3 / (s + 1e-6)).reshape(RT, NB))

    rep = rep_ref[...]
    mask = mask_ref[...]
    for ch in range(RT // 256):
        pc = perm[ch * 256:(ch + 1) * 256, :]
        bd = _bf(lax.dot_general(pc, rep, (((1,), (0,)), ((), ())),
                                 preferred_element_type=jnp.float32)) * mask
        rc = rhs[ch * 256:(ch + 1) * 256, :]
        gtc = lax.dot_general(bd, rc, (((0,), (0,)), ((), ())),
                              preferred_element_type=jnp.float32)
        acc_ref[ch * 256:(ch + 1) * 256, :] = _bf(gtc)


def _k1b_body(gflat_ref, w2_ref, convb_ref, outt_ref, psum_ref, psumsq_ref):
    outt = lax.dot_general(w2_ref[...], gflat_ref[...], (((0,), (1,)), ((), ())),
                           preferred_element_type=jnp.float32)
    outt = outt + convb_ref[...]
    outt_ref[...] = outt
    psum_ref[0] = jnp.sum(outt, axis=1, keepdims=True)
    psumsq_ref[0] = jnp.sum(outt * outt, axis=1, keepdims=True)


def _k2_body(pre_ref, psum_ref, psumsq_ref, gamma_ref, beta_ref, out_ref, *, total):
    s = jnp.sum(psum_ref[...], axis=0)
    ss = jnp.sum(psumsq_ref[...], axis=0)
    mean = s / total
    var = ss / total - mean * mean
    scale = gamma_ref[...] * lax.rsqrt(var + 1e-5)
    shift = beta_ref[...] - mean * scale
    out_ref[0] = pre_ref[...] * scale + shift


def kernel(x, feature, neigh_indexs, B, kernels, mlp_w, mlp_b, conv_w, conv_b, bn_gamma, bn_beta):
    BS, C, N = feature.shape
    P = BS * N
    R = P * NB
    T = P // PT

    ftab = jnp.transpose(feature, (0, 2, 1)).reshape(P, C).astype(jnp.bfloat16)
    ctab = jnp.concatenate(
        [jnp.transpose(x, (0, 2, 1)).reshape(P, 3), jnp.zeros((P, 13), jnp.float32)], axis=1)
    base = (jnp.arange(BS, dtype=jnp.int32) * N).reshape(BS, 1, 1)
    ni32 = neigh_indexs.astype(jnp.int32)
    flat_ni = (ni32 + base).reshape(-1)
    flat_ni0 = jnp.broadcast_to(ni32[:, :, 0:1] + base, (BS, N, NB)).reshape(-1)

    zero_row = jnp.zeros((1, 32), jnp.float32)
    b8 = jnp.concatenate([B[0:3], zero_row, B[3:6], zero_row], axis=0).astype(jnp.bfloat16)
    b6 = B[6:7].astype(jnp.bfloat16).astype(jnp.float32)
    k4 = jnp.concatenate([kernels, zero_row], axis=0).astype(jnp.bfloat16)
    mlpwt = mlp_w.T.astype(jnp.bfloat16)
    mlpb = mlp_b.reshape(1, C_X)
    w2 = conv_w.reshape(64, C_ALL, NB).transpose(2, 1, 0).reshape(NB * C_ALL, 64).astype(jnp.bfloat16)
    convb = conv_b.reshape(64, 1)
    onep = jnp.zeros((NB, NB), jnp.float32).at[0, 0].set(1.0)
    ji = jnp.arange(NB, dtype=jnp.int32)
    ci = jnp.arange(256, dtype=jnp.int32)
    rep = ((ci[None, :] % NB) == ji[:, None]).astype(jnp.bfloat16)
    bdmask = (ci[:, None] // NB == ci[None, :] // NB).astype(jnp.bfloat16)
    gamma = bn_gamma.reshape(64, 1)
    beta = bn_beta.reshape(64, 1)

    gf, gx, gx0 = _make_sc_gather(R, P, C)(flat_ni, flat_ni0, ftab, ctab)

    full = lambda shape: pl.BlockSpec(shape, lambda t: (0,) * len(shape))
    gt = pl.pallas_call(
        _k1_body,
        grid=(T,),
        in_specs=[
            pl.BlockSpec((RT, C_IN), lambda t: (t, 0)),
            pl.BlockSpec((RT, 16), lambda t: (t, 0)),
            pl.BlockSpec((RT, 16), lambda t: (t, 0)),
            full((8, 32)), full((1, 32)), full((4, 32)),
            full((64, C_X)), full((1, C_X)), full((NB, NB)), full((NB, 256)),
            full((256, 256)),
        ],
        out_specs=pl.BlockSpec((RT, C_ALL), lambda t: (t, 0)),
        out_shape=jax.ShapeDtypeStruct((R, C_ALL), jnp.bfloat16),
    )(gf, gx, gx0, b8, b6, k4, mlpwt, mlpb, onep, rep, bdmask)

    gflat = gt.reshape(P, NB * C_ALL)

    outt, psum, psumsq = pl.pallas_call(
        _k1b_body,
        grid=(T,),
        in_specs=[
            pl.BlockSpec((PT, NB * C_ALL), lambda t: (t, 0)),
            full((NB * C_ALL, 64)), full((64, 1)),
        ],
        out_specs=[
            pl.BlockSpec((64, PT), lambda t: (0, t)),
            pl.BlockSpec((1, 64, 1), lambda t: (t, 0, 0)),
            pl.BlockSpec((1, 64, 1), lambda t: (t, 0, 0)),
        ],
        out_shape=[
            jax.ShapeDtypeStruct((64, P), jnp.float32),
            jax.ShapeDtypeStruct((T, 64, 1), jnp.float32),
            jax.ShapeDtypeStruct((T, 64, 1), jnp.float32),
        ],
    )(gflat, w2, convb)

    NT2 = 1024
    T2 = P // NT2
    out = pl.pallas_call(
        functools.partial(_k2_body, total=float(P)),
        grid=(T2,),
        in_specs=[
            pl.BlockSpec((64, NT2), lambda t: (0, t)),
            pl.BlockSpec((T, 64, 1), lambda t: (0, 0, 0)),
            pl.BlockSpec((T, 64, 1), lambda t: (0, 0, 0)),
            pl.BlockSpec((64, 1), lambda t: (0, 0)),
            pl.BlockSpec((64, 1), lambda t: (0, 0)),
        ],
        out_specs=pl.BlockSpec((1, 64, NT2), lambda t: (t * NT2 // N, 0, (t * NT2 % N) // NT2)),
        out_shape=jax.ShapeDtypeStruct((BS, 64, N), jnp.float32),
    )(outt, psum, psumsq, gamma, beta)
    return out
